# baseline (speedup 1.0000x reference)
import functools

import jax
import jax.numpy as jnp
from jax import lax
from jax.experimental import pallas as pl
from jax.experimental.pallas import tpu as pltpu
from jax.experimental.pallas import tpu_sc as plsc


_G = 16
_HK = 16
_NW = 32

_SC_B = 384
_BB = 64


def _sc_pair_loop(xb, t2b, ob, neg_lam_v, n_pairs, d):
    nch = d // 16

    def pair_body(p, _):
        w = jnp.exp(neg_lam_v * t2b[p])
        s = w[0]
        for k in range(1, _HK):
            s = s + w[k]
        totv = jnp.broadcast_to(s, (16,))
        accs = [jnp.zeros((16,), jnp.float32) for _ in range(nch)]
        for k in range(_HK):
            wk = w[k]
            for c in range(nch):
                accs[c] = accs[c] + wk * xb[p * _HK + k, pl.ds(c * 16, 16)]
        for c in range(nch):
            ob[p, pl.ds(c * 16, 16)] = accs[c] / totv
        return 0

    lax.fori_loop(0, n_pairs, pair_body, 0, unroll=4)


def _sc_reduce_body(x_hbm, t2_hbm, lam_hbm, out_hbm,
                    xb0, xb1, t2b0, t2b1, ob0, ob1, lamv,
                    sem_x0, sem_x1, sem_t0, sem_t1, sem_o0, sem_o1,
                    *, n_pairs_tot):
    _, d = x_hbm.shape
    per_w = n_pairs_tot // _NW
    ngrp = per_w // _G

    wid = lax.axis_index("s") * 2 + lax.axis_index("c")
    base = wid * per_w

    pltpu.sync_copy(lam_hbm, lamv)
    neg_lam_v = -lamv[...]

    xbs = (xb0, xb1)
    t2bs = (t2b0, t2b1)
    obs = (ob0, ob1)
    sxs = (sem_x0, sem_x1)
    sts = (sem_t0, sem_t1)
    sos = (sem_o0, sem_o1)

    def start_in(g, sl):
        row0 = (base + g * _G) * _HK
        pltpu.async_copy(x_hbm.at[pl.ds(row0, _G * _HK)], xbs[sl], sxs[sl])
        pltpu.async_copy(t2_hbm.at[pl.ds(base + g * _G, _G)], t2bs[sl], sts[sl])

    def wait_in(sl):
        pltpu.make_async_copy(x_hbm.at[pl.ds(0, _G * _HK)], xbs[sl], sxs[sl]).wait()
        pltpu.make_async_copy(t2_hbm.at[pl.ds(0, _G)], t2bs[sl], sts[sl]).wait()

    start_in(0, 0)

    def outer(i, _):
        for sl in range(2):
            g = i * 2 + sl

            @pl.when(g + 1 < ngrp)
            def _():
                start_in(g + 1, 1 - sl)

            wait_in(sl)

            @pl.when(g >= 2)
            def _():
                pltpu.make_async_copy(
                    obs[sl], out_hbm.at[pl.ds(base + (g - 2) * _G, _G)],
                    sos[sl]).wait()

            _sc_pair_loop(xbs[sl], t2bs[sl], obs[sl], neg_lam_v, _G, d)
            pltpu.async_copy(obs[sl], out_hbm.at[pl.ds(base + g * _G, _G)],
                             sos[sl])
        return 0

    lax.fori_loop(0, ngrp // 2, outer, 0)
    for sl in range(2):
        pltpu.make_async_copy(
            obs[sl], out_hbm.at[pl.ds(base + (ngrp - 2 + sl) * _G, _G)],
            sos[sl]).wait()


def _sc_weighted_reduce(two_flat, t2_flat, lam16, n_pairs_head):
    n_rows, d = two_flat.shape
    mesh = plsc.VectorSubcoreMesh(core_axis_name="c", subcore_axis_name="s")
    return pl.kernel(
        functools.partial(_sc_reduce_body, n_pairs_tot=n_pairs_head),
        mesh=mesh,
        out_type=jax.ShapeDtypeStruct((n_rows // _HK, d), jnp.float32),
        scratch_types=[
            pltpu.VMEM((_G * _HK, d), jnp.float32),
            pltpu.VMEM((_G * _HK, d), jnp.float32),
            pltpu.VMEM((_G, _HK), jnp.float32),
            pltpu.VMEM((_G, _HK), jnp.float32),
            pltpu.VMEM((_G, d), jnp.float32),
            pltpu.VMEM((_G, d), jnp.float32),
            pltpu.VMEM((16,), jnp.float32),
            pltpu.SemaphoreType.DMA,
            pltpu.SemaphoreType.DMA,
            pltpu.SemaphoreType.DMA,
            pltpu.SemaphoreType.DMA,
            pltpu.SemaphoreType.DMA,
            pltpu.SemaphoreType.DMA,
        ],
    )(two_flat, t2_flat, lam16)


def _branch_emb(self_ref, one_ref, two_ref, t1_ref, t2_ref, W0, W1, W2, lam):
    BB, H, _, D = two_ref.shape
    HID = W0.shape[1]
    t2 = t2_ref[...]
    a2 = jax.nn.softmax(-lam * t2, axis=-1)
    two = two_ref[...]
    red = jnp.sum(a2[..., None] * two, axis=2)
    return _finish_emb(self_ref, one_ref, red.reshape(BB * H, D), t1_ref,
                       W0, W1, W2, lam, BB, H, D, HID)


def _finish_emb(self_ref, one_ref, red2, t1_ref, W0, W1, W2, lam, BB, H, D, HID):
    m2 = jnp.dot(red2, W2, preferred_element_type=jnp.float32)
    one2 = one_ref[...].reshape(BB * H, D)
    h1 = jnp.tanh(jnp.dot(one2, W1, preferred_element_type=jnp.float32) + m2)
    t1 = t1_ref[...]
    a1 = jax.nn.softmax(-lam * t1, axis=-1)
    m1 = jnp.sum(a1[..., None] * h1.reshape(BB, H, HID), axis=1)
    return jnp.tanh(jnp.dot(self_ref[...], W0, preferred_element_type=jnp.float32) + m1)


def _cos_rows(a, b):
    num = jnp.sum(a * b, axis=-1, keepdims=True)
    na = jnp.sqrt(jnp.sum(a * a, axis=-1, keepdims=True))
    nb = jnp.sqrt(jnp.sum(b * b, axis=-1, keepdims=True))
    return num / jnp.maximum(na * nb, 1e-8)


def _loss_terms(s_emb, t_emb, n_emb, str_blk, snr_blk):
    pos = (1.0 - _cos_rows(s_emb, t_emb)) * str_blk
    cn = _cos_rows(s_emb, n_emb)
    eterm = jnp.where(snr_blk > 0.0, 1.0 - cn, jnp.maximum(cn, 0.0))
    return jnp.sum(pos), jnp.sum(eterm)


def _tc_st_tail_body(lam_ref, str_ref, snr_ref, W0_ref, W1_ref, W2_ref,
                     s_self, s_one, s_two, s_t1, s_t2,
                     t_self, t_one, t_two, t_t1, t_t2,
                     n_self, n_one, n_two, n_t1, n_t2,
                     s_out, t_out, n_out, sums_out, acc_ref, *, hb):
    lam = lam_ref[0, 0]
    W0 = W0_ref[...]
    W1 = W1_ref[...]
    W2 = W2_ref[...]
    s_emb = _branch_emb(s_self, s_one, s_two, s_t1, s_t2, W0, W1, W2, lam)
    t_emb = _branch_emb(t_self, t_one, t_two, t_t1, t_t2, W0, W1, W2, lam)
    s_out[...] = s_emb
    t_out[...] = t_emb

    @pl.when(pl.program_id(0) == 0)
    def _init():
        acc_ref[0] = 0.0
        acc_ref[1] = 0.0

    @pl.when(pl.program_id(0) >= hb)
    def _tail():
        n_emb = _branch_emb(n_self, n_one, n_two, n_t1, n_t2, W0, W1, W2, lam)
        n_out[...] = n_emb
        ps, es = _loss_terms(s_emb, t_emb, n_emb, str_ref[...], snr_ref[...])
        acc_ref[0] += ps
        acc_ref[1] += es

    @pl.when(pl.program_id(0) == pl.num_programs(0) - 1)
    def _fin():
        sums_out[0] = acc_ref[0]
        sums_out[1] = acc_ref[1]


def _tc_neg_head_body(lam_ref, tail_sums_ref, str_ref, snr_ref,
                      W0_ref, W1_ref, W2_ref,
                      n_self, n_one, n_red, n_t1,
                      s_emb_ref, t_emb_ref,
                      n_out, L_out, acc_ref, *, total_b):
    lam = lam_ref[0, 0]
    BB, H, D = n_one.shape
    HID = W0_ref.shape[1]
    n_emb = _finish_emb(n_self, n_one, n_red[...].reshape(BB * H, D), n_t1,
                        W0_ref[...], W1_ref[...], W2_ref[...], lam, BB, H, D, HID)
    n_out[...] = n_emb

    ps, es = _loss_terms(s_emb_ref[...], t_emb_ref[...], n_emb,
                         str_ref[...], snr_ref[...])

    @pl.when(pl.program_id(0) == 0)
    def _init():
        acc_ref[0] = 0.0
        acc_ref[1] = 0.0

    acc_ref[0] += ps
    acc_ref[1] += es

    @pl.when(pl.program_id(0) == pl.num_programs(0) - 1)
    def _fin():
        lpos = (acc_ref[0] + tail_sums_ref[0]) / total_b
        el = (acc_ref[1] + tail_sums_ref[1]) / total_b
        L_out[...] = jnp.full((1, 1), lpos + el * lpos, dtype=jnp.float32)


def kernel(s_self_feat, s_one_hop_feat, s_two_hop_feat,
           t_self_feat, t_one_hop_feat, t_two_hop_feat,
           neg_self_feat, neg_one_hop_feat, neg_two_hop_feat,
           s_his_time, s_his_his_time, t_his_time, t_his_his_time,
           neg_his_time, neg_his_his_time,
           s_edge_rate, s_t_rate, s_n_rate, W0, W1, W2, lam, training=False):
    B, H, D = s_one_hop_feat.shape
    HID = W0.shape[1]
    BB = _BB
    SCB = _SC_B
    head_blocks = SCB // BB
    tail_blocks = (B - SCB) // BB

    lam2 = jnp.reshape(lam, (1, 1))
    str2 = jnp.reshape(s_t_rate, (B, 1))
    snr2 = jnp.reshape(s_n_rate, (B, 1))
    neg_t1 = jnp.reshape(neg_his_time, (B, H))

    lam16 = jnp.full((16,), lam, dtype=jnp.float32)
    n_red = _sc_weighted_reduce(
        neg_two_hop_feat.reshape(B * H * H, D),
        neg_his_his_time.reshape(B * H, H),
        lam16,
        SCB * H,
    ).reshape(B, H, D)

    spec_w = pl.BlockSpec((D, HID), lambda i: (0, 0))
    spec_self = pl.BlockSpec((BB, D), lambda i: (i, 0))
    spec_one = pl.BlockSpec((BB, H, D), lambda i: (i, 0, 0))
    spec_two = pl.BlockSpec((BB, H, H, D), lambda i: (i, 0, 0, 0))
    spec_t1 = pl.BlockSpec((BB, H), lambda i: (i, 0))
    spec_t2 = pl.BlockSpec((BB, H, H), lambda i: (i, 0, 0))
    spec_rate = pl.BlockSpec((BB, 1), lambda i: (i, 0))
    spec_emb = pl.BlockSpec((BB, HID), lambda i: (i, 0))

    hb = head_blocks
    spec_self_c = pl.BlockSpec((BB, D), lambda i: (jnp.maximum(i, hb), 0))
    spec_one_c = pl.BlockSpec((BB, H, D), lambda i: (jnp.maximum(i, hb), 0, 0))
    spec_two_c = pl.BlockSpec((BB, H, H, D), lambda i: (jnp.maximum(i, hb), 0, 0, 0))
    spec_t1_c = pl.BlockSpec((BB, H), lambda i: (jnp.maximum(i, hb), 0))
    spec_t2_c = pl.BlockSpec((BB, H, H), lambda i: (jnp.maximum(i, hb), 0, 0))
    spec_rate_c = pl.BlockSpec((BB, 1), lambda i: (jnp.maximum(i, hb), 0))
    spec_emb_tail = pl.BlockSpec((BB, HID), lambda i: (jnp.maximum(i - hb, 0), 0))

    s_emb, t_emb, n_emb_tail, tail_sums = pl.pallas_call(
        functools.partial(_tc_st_tail_body, hb=hb),
        grid=(B // BB,),
        in_specs=[
            pl.BlockSpec(memory_space=pltpu.SMEM),
            spec_rate_c, spec_rate_c,
            spec_w, spec_w, spec_w,
            spec_self, spec_one, spec_two, spec_t1, spec_t2,
            spec_self, spec_one, spec_two, spec_t1, spec_t2,
            spec_self_c, spec_one_c, spec_two_c, spec_t1_c, spec_t2_c,
        ],
        out_specs=[spec_emb, spec_emb, spec_emb_tail,
                   pl.BlockSpec(memory_space=pltpu.SMEM)],
        out_shape=[
            jax.ShapeDtypeStruct((B, HID), jnp.float32),
            jax.ShapeDtypeStruct((B, HID), jnp.float32),
            jax.ShapeDtypeStruct((B - SCB, HID), jnp.float32),
            jax.ShapeDtypeStruct((2,), jnp.float32),
        ],
        scratch_shapes=[pltpu.SMEM((2,), jnp.float32)],
    )(lam2, str2, snr2, W0, W1, W2,
      s_self_feat, s_one_hop_feat, s_two_hop_feat, s_his_time, s_his_his_time,
      t_self_feat, t_one_hop_feat, t_two_hop_feat, t_his_time, t_his_his_time,
      neg_self_feat, neg_one_hop_feat, neg_two_hop_feat, neg_t1,
      neg_his_his_time)

    n_emb_head, L = pl.pallas_call(
        functools.partial(_tc_neg_head_body, total_b=float(B)),
        grid=(head_blocks,),
        in_specs=[
            pl.BlockSpec(memory_space=pltpu.SMEM),
            pl.BlockSpec(memory_space=pltpu.SMEM),
            spec_rate, spec_rate,
            spec_w, spec_w, spec_w,
            spec_self, spec_one, spec_one, spec_t1,
            spec_emb, spec_emb,
        ],
        out_specs=[spec_emb, pl.BlockSpec((1, 1), lambda i: (0, 0))],
        out_shape=[
            jax.ShapeDtypeStruct((SCB, HID), jnp.float32),
            jax.ShapeDtypeStruct((1, 1), jnp.float32),
        ],
        scratch_shapes=[pltpu.SMEM((2,), jnp.float32)],
    )(lam2, tail_sums, str2, snr2, W0, W1, W2,
      neg_self_feat, neg_one_hop_feat, n_red, neg_t1,
      s_emb, t_emb)

    n_emb = jnp.concatenate([n_emb_head, n_emb_tail], axis=0)
    L0 = L[0, 0]
    return (L0, s_emb, t_emb, s_emb, n_emb)


# speedup vs baseline: 1.0134x; 1.0134x over previous
import functools

import jax
import jax.numpy as jnp
from jax import lax
from jax.experimental import pallas as pl
from jax.experimental.pallas import tpu as pltpu
from jax.experimental.pallas import tpu_sc as plsc


_G = 16
_HK = 16
_NW = 32

_SC_B = 256
_BB = 64


def _sc_pair_loop(xb, t2b, ob, neg_lam_v, n_pairs, d):
    nch = d // 16

    def pair_body(p, _):
        w = jnp.exp(neg_lam_v * t2b[p])
        s = w[0]
        for k in range(1, _HK):
            s = s + w[k]
        totv = jnp.broadcast_to(s, (16,))
        accs = [jnp.zeros((16,), jnp.float32) for _ in range(nch)]
        for k in range(_HK):
            wk = w[k]
            for c in range(nch):
                accs[c] = accs[c] + wk * xb[p * _HK + k, pl.ds(c * 16, 16)]
        for c in range(nch):
            ob[p, pl.ds(c * 16, 16)] = accs[c] / totv
        return 0

    lax.fori_loop(0, n_pairs, pair_body, 0, unroll=4)


def _sc_reduce_body(x_hbm, t2_hbm, lam_hbm, out_hbm,
                    xb0, xb1, t2b0, t2b1, ob0, ob1, lamv,
                    sem_x0, sem_x1, sem_t0, sem_t1, sem_o0, sem_o1,
                    *, n_pairs_tot):
    _, d = x_hbm.shape
    per_w = n_pairs_tot // _NW
    ngrp = per_w // _G

    wid = lax.axis_index("s") * 2 + lax.axis_index("c")
    base = wid * per_w

    pltpu.sync_copy(lam_hbm, lamv)
    neg_lam_v = -lamv[...]

    xbs = (xb0, xb1)
    t2bs = (t2b0, t2b1)
    obs = (ob0, ob1)
    sxs = (sem_x0, sem_x1)
    sts = (sem_t0, sem_t1)
    sos = (sem_o0, sem_o1)

    def start_in(g, sl):
        row0 = (base + g * _G) * _HK
        pltpu.async_copy(x_hbm.at[pl.ds(row0, _G * _HK)], xbs[sl], sxs[sl])
        pltpu.async_copy(t2_hbm.at[pl.ds(base + g * _G, _G)], t2bs[sl], sts[sl])

    def wait_in(sl):
        pltpu.make_async_copy(x_hbm.at[pl.ds(0, _G * _HK)], xbs[sl], sxs[sl]).wait()
        pltpu.make_async_copy(t2_hbm.at[pl.ds(0, _G)], t2bs[sl], sts[sl]).wait()

    start_in(0, 0)

    def outer(i, _):
        for sl in range(2):
            g = i * 2 + sl

            @pl.when(g + 1 < ngrp)
            def _():
                start_in(g + 1, 1 - sl)

            wait_in(sl)

            @pl.when(g >= 2)
            def _():
                pltpu.make_async_copy(
                    obs[sl], out_hbm.at[pl.ds(base + (g - 2) * _G, _G)],
                    sos[sl]).wait()

            _sc_pair_loop(xbs[sl], t2bs[sl], obs[sl], neg_lam_v, _G, d)
            pltpu.async_copy(obs[sl], out_hbm.at[pl.ds(base + g * _G, _G)],
                             sos[sl])
        return 0

    lax.fori_loop(0, ngrp // 2, outer, 0)
    for sl in range(2):
        pltpu.make_async_copy(
            obs[sl], out_hbm.at[pl.ds(base + (ngrp - 2 + sl) * _G, _G)],
            sos[sl]).wait()


def _sc_weighted_reduce(two_flat, t2_flat, lam16, n_pairs_head):
    n_rows, d = two_flat.shape
    mesh = plsc.VectorSubcoreMesh(core_axis_name="c", subcore_axis_name="s")
    return pl.kernel(
        functools.partial(_sc_reduce_body, n_pairs_tot=n_pairs_head),
        mesh=mesh,
        out_type=jax.ShapeDtypeStruct((n_rows // _HK, d), jnp.float32),
        scratch_types=[
            pltpu.VMEM((_G * _HK, d), jnp.float32),
            pltpu.VMEM((_G * _HK, d), jnp.float32),
            pltpu.VMEM((_G, _HK), jnp.float32),
            pltpu.VMEM((_G, _HK), jnp.float32),
            pltpu.VMEM((_G, d), jnp.float32),
            pltpu.VMEM((_G, d), jnp.float32),
            pltpu.VMEM((16,), jnp.float32),
            pltpu.SemaphoreType.DMA,
            pltpu.SemaphoreType.DMA,
            pltpu.SemaphoreType.DMA,
            pltpu.SemaphoreType.DMA,
            pltpu.SemaphoreType.DMA,
            pltpu.SemaphoreType.DMA,
        ],
    )(two_flat, t2_flat, lam16)


def _branch_emb(self_ref, one_ref, two_ref, t1_ref, t2_ref, W0, W1, W2, lam):
    BB, H, _, D = two_ref.shape
    HID = W0.shape[1]
    t2 = t2_ref[...]
    a2 = jax.nn.softmax(-lam * t2, axis=-1)
    two = two_ref[...]
    red = jnp.sum(a2[..., None] * two, axis=2)
    return _finish_emb(self_ref, one_ref, red.reshape(BB * H, D), t1_ref,
                       W0, W1, W2, lam, BB, H, D, HID)


def _finish_emb(self_ref, one_ref, red2, t1_ref, W0, W1, W2, lam, BB, H, D, HID):
    m2 = jnp.dot(red2, W2, preferred_element_type=jnp.float32)
    one2 = one_ref[...].reshape(BB * H, D)
    h1 = jnp.tanh(jnp.dot(one2, W1, preferred_element_type=jnp.float32) + m2)
    t1 = t1_ref[...]
    a1 = jax.nn.softmax(-lam * t1, axis=-1)
    m1 = jnp.sum(a1[..., None] * h1.reshape(BB, H, HID), axis=1)
    return jnp.tanh(jnp.dot(self_ref[...], W0, preferred_element_type=jnp.float32) + m1)


def _cos_rows(a, b):
    num = jnp.sum(a * b, axis=-1, keepdims=True)
    na = jnp.sqrt(jnp.sum(a * a, axis=-1, keepdims=True))
    nb = jnp.sqrt(jnp.sum(b * b, axis=-1, keepdims=True))
    return num / jnp.maximum(na * nb, 1e-8)


def _loss_terms(s_emb, t_emb, n_emb, str_blk, snr_blk):
    pos = (1.0 - _cos_rows(s_emb, t_emb)) * str_blk
    cn = _cos_rows(s_emb, n_emb)
    eterm = jnp.where(snr_blk > 0.0, 1.0 - cn, jnp.maximum(cn, 0.0))
    return jnp.sum(pos), jnp.sum(eterm)


def _tc_st_tail_body(lam_ref, str_ref, snr_ref, W0_ref, W1_ref, W2_ref,
                     s_self, s_one, s_two, s_t1, s_t2,
                     t_self, t_one, t_two, t_t1, t_t2,
                     n_self, n_one, n_two, n_t1, n_t2,
                     s_out, t_out, n_out, sums_out, acc_ref, *, hb):
    lam = lam_ref[0, 0]
    W0 = W0_ref[...]
    W1 = W1_ref[...]
    W2 = W2_ref[...]
    s_emb = _branch_emb(s_self, s_one, s_two, s_t1, s_t2, W0, W1, W2, lam)
    t_emb = _branch_emb(t_self, t_one, t_two, t_t1, t_t2, W0, W1, W2, lam)
    s_out[...] = s_emb
    t_out[...] = t_emb

    @pl.when(pl.program_id(0) == 0)
    def _init():
        acc_ref[0] = 0.0
        acc_ref[1] = 0.0

    @pl.when(pl.program_id(0) >= hb)
    def _tail():
        n_emb = _branch_emb(n_self, n_one, n_two, n_t1, n_t2, W0, W1, W2, lam)
        n_out[...] = n_emb
        ps, es = _loss_terms(s_emb, t_emb, n_emb, str_ref[...], snr_ref[...])
        acc_ref[0] += ps
        acc_ref[1] += es

    @pl.when(pl.program_id(0) == pl.num_programs(0) - 1)
    def _fin():
        sums_out[0] = acc_ref[0]
        sums_out[1] = acc_ref[1]


def _tc_neg_head_body(lam_ref, tail_sums_ref, str_ref, snr_ref,
                      W0_ref, W1_ref, W2_ref,
                      n_self, n_one, n_red, n_t1,
                      s_emb_ref, t_emb_ref,
                      n_out, L_out, acc_ref, *, total_b):
    lam = lam_ref[0, 0]
    BB, H, D = n_one.shape
    HID = W0_ref.shape[1]
    n_emb = _finish_emb(n_self, n_one, n_red[...].reshape(BB * H, D), n_t1,
                        W0_ref[...], W1_ref[...], W2_ref[...], lam, BB, H, D, HID)
    n_out[...] = n_emb

    ps, es = _loss_terms(s_emb_ref[...], t_emb_ref[...], n_emb,
                         str_ref[...], snr_ref[...])

    @pl.when(pl.program_id(0) == 0)
    def _init():
        acc_ref[0] = 0.0
        acc_ref[1] = 0.0

    acc_ref[0] += ps
    acc_ref[1] += es

    @pl.when(pl.program_id(0) == pl.num_programs(0) - 1)
    def _fin():
        lpos = (acc_ref[0] + tail_sums_ref[0]) / total_b
        el = (acc_ref[1] + tail_sums_ref[1]) / total_b
        L_out[...] = jnp.full((1, 1), lpos + el * lpos, dtype=jnp.float32)


def kernel(s_self_feat, s_one_hop_feat, s_two_hop_feat,
           t_self_feat, t_one_hop_feat, t_two_hop_feat,
           neg_self_feat, neg_one_hop_feat, neg_two_hop_feat,
           s_his_time, s_his_his_time, t_his_time, t_his_his_time,
           neg_his_time, neg_his_his_time,
           s_edge_rate, s_t_rate, s_n_rate, W0, W1, W2, lam, training=False):
    B, H, D = s_one_hop_feat.shape
    HID = W0.shape[1]
    BB = _BB
    SCB = _SC_B
    head_blocks = SCB // BB

    lam2 = jnp.reshape(lam, (1, 1))
    str2 = jnp.reshape(s_t_rate, (B, 1))
    snr2 = jnp.reshape(s_n_rate, (B, 1))
    neg_t1 = jnp.reshape(neg_his_time, (B, H))

    lam16 = jnp.full((16,), lam, dtype=jnp.float32)
    n_red = _sc_weighted_reduce(
        neg_two_hop_feat.reshape(B * H * H, D),
        neg_his_his_time.reshape(B * H, H),
        lam16,
        SCB * H,
    ).reshape(B, H, D)

    spec_w = pl.BlockSpec((D, HID), lambda i: (0, 0))
    spec_self = pl.BlockSpec((BB, D), lambda i: (i, 0))
    spec_one = pl.BlockSpec((BB, H, D), lambda i: (i, 0, 0))
    spec_two = pl.BlockSpec((BB, H, H, D), lambda i: (i, 0, 0, 0))
    spec_t1 = pl.BlockSpec((BB, H), lambda i: (i, 0))
    spec_t2 = pl.BlockSpec((BB, H, H), lambda i: (i, 0, 0))
    spec_rate = pl.BlockSpec((BB, 1), lambda i: (i, 0))
    spec_emb = pl.BlockSpec((BB, HID), lambda i: (i, 0))

    hb = head_blocks
    spec_self_c = pl.BlockSpec((BB, D), lambda i: (jnp.maximum(i, hb), 0))
    spec_one_c = pl.BlockSpec((BB, H, D), lambda i: (jnp.maximum(i, hb), 0, 0))
    spec_two_c = pl.BlockSpec((BB, H, H, D), lambda i: (jnp.maximum(i, hb), 0, 0, 0))
    spec_t1_c = pl.BlockSpec((BB, H), lambda i: (jnp.maximum(i, hb), 0))
    spec_t2_c = pl.BlockSpec((BB, H, H), lambda i: (jnp.maximum(i, hb), 0, 0))
    spec_rate_c = pl.BlockSpec((BB, 1), lambda i: (jnp.maximum(i, hb), 0))
    spec_emb_tail = pl.BlockSpec((BB, HID), lambda i: (jnp.maximum(i - hb, 0), 0))

    s_emb, t_emb, n_emb_tail, tail_sums = pl.pallas_call(
        functools.partial(_tc_st_tail_body, hb=hb),
        grid=(B // BB,),
        in_specs=[
            pl.BlockSpec(memory_space=pltpu.SMEM),
            spec_rate_c, spec_rate_c,
            spec_w, spec_w, spec_w,
            spec_self, spec_one, spec_two, spec_t1, spec_t2,
            spec_self, spec_one, spec_two, spec_t1, spec_t2,
            spec_self_c, spec_one_c, spec_two_c, spec_t1_c, spec_t2_c,
        ],
        out_specs=[spec_emb, spec_emb, spec_emb_tail,
                   pl.BlockSpec(memory_space=pltpu.SMEM)],
        out_shape=[
            jax.ShapeDtypeStruct((B, HID), jnp.float32),
            jax.ShapeDtypeStruct((B, HID), jnp.float32),
            jax.ShapeDtypeStruct((B - SCB, HID), jnp.float32),
            jax.ShapeDtypeStruct((2,), jnp.float32),
        ],
        scratch_shapes=[pltpu.SMEM((2,), jnp.float32)],
    )(lam2, str2, snr2, W0, W1, W2,
      s_self_feat, s_one_hop_feat, s_two_hop_feat, s_his_time, s_his_his_time,
      t_self_feat, t_one_hop_feat, t_two_hop_feat, t_his_time, t_his_his_time,
      neg_self_feat, neg_one_hop_feat, neg_two_hop_feat, neg_t1,
      neg_his_his_time)

    n_emb_head, L = pl.pallas_call(
        functools.partial(_tc_neg_head_body, total_b=float(B)),
        grid=(head_blocks,),
        in_specs=[
            pl.BlockSpec(memory_space=pltpu.SMEM),
            pl.BlockSpec(memory_space=pltpu.SMEM),
            spec_rate, spec_rate,
            spec_w, spec_w, spec_w,
            spec_self, spec_one, spec_one, spec_t1,
            spec_emb, spec_emb,
        ],
        out_specs=[spec_emb, pl.BlockSpec((1, 1), lambda i: (0, 0))],
        out_shape=[
            jax.ShapeDtypeStruct((SCB, HID), jnp.float32),
            jax.ShapeDtypeStruct((1, 1), jnp.float32),
        ],
        scratch_shapes=[pltpu.SMEM((2,), jnp.float32)],
    )(lam2, tail_sums, str2, snr2, W0, W1, W2,
      neg_self_feat, neg_one_hop_feat, n_red, neg_t1,
      s_emb, t_emb)

    n_emb = jnp.concatenate([n_emb_head, n_emb_tail], axis=0)
    L0 = L[0, 0]
    return (L0, s_emb, t_emb, s_emb, n_emb)
